# Initial kernel scaffold; baseline (speedup 1.0000x reference)
import jax
import jax.numpy as jnp
from jax.experimental import pallas as pl


def kernel(image, perm):
    raise NotImplementedError("write your pallas kernel here")


# speedup vs baseline: 1.7703x; 1.7703x over previous
import functools

import jax
import jax.numpy as jnp
from jax import lax
from jax.experimental import pallas as pl
from jax.experimental.pallas import tpu as pltpu
from jax.experimental.pallas import tpu_sc as plsc

_LANES = 16


@functools.lru_cache(maxsize=None)
def _build(P, C, NP):
    info = plsc.get_sparse_core_info()
    NC, NS = info.num_cores, info.num_subcores
    NW = NC * NS
    assert P % (NW * NP) == 0 and C % _LANES == 0
    rows_per_worker = P // NW
    nblocks = rows_per_worker // NP
    nchunks = C // _LANES
    BLK = NP * C

    mesh = plsc.VectorSubcoreMesh(core_axis_name="c", subcore_axis_name="s")

    @functools.partial(
        pl.kernel,
        mesh=mesh,
        compiler_params=pltpu.CompilerParams(needs_layout_passes=False),
        out_type=jax.ShapeDtypeStruct((P * C,), jnp.float32),
        scratch_types=[
            pltpu.VMEM((BLK,), jnp.float32),
            pltpu.VMEM((BLK,), jnp.int32),
            pltpu.VMEM((BLK,), jnp.float32),
        ],
    )
    def k(img_hbm, perm_hbm, out_hbm, img_v, perm_v, out_v):
        wid = lax.axis_index("s") * NC + lax.axis_index("c")
        base = wid * rows_per_worker * C

        def block(b, carry):
            elem0 = base + b * BLK
            pltpu.sync_copy(img_hbm.at[pl.ds(elem0, BLK)], img_v)
            pltpu.sync_copy(perm_hbm.at[pl.ds(elem0, BLK)], perm_v)

            def pix(i, c):
                basei = i * C
                off = jnp.full((_LANES,), basei, jnp.int32)
                for j in range(nchunks):
                    idx = perm_v[pl.ds(basei + j * _LANES, _LANES)]
                    val = plsc.load_gather(img_v, [idx + off])
                    out_v[pl.ds(basei + j * _LANES, _LANES)] = val
                return c

            lax.fori_loop(0, NP, pix, 0)
            pltpu.sync_copy(out_v, out_hbm.at[pl.ds(elem0, BLK)])
            return carry

        lax.fori_loop(0, nblocks, block, 0)

    return k


def kernel(image, perm):
    W, H, C = image.shape
    P = W * H
    img1 = image.reshape(P * C)
    perm1 = perm.reshape(P * C)
    out1 = _build(P, C, 32)(img1, perm1)
    return out1.reshape(W, H, C)


# speedup vs baseline: 2.1810x; 1.2320x over previous
import functools

import jax
import jax.numpy as jnp
from jax import lax
from jax.experimental import pallas as pl
from jax.experimental.pallas import tpu as pltpu
from jax.experimental.pallas import tpu_sc as plsc

_LANES = 16


@functools.lru_cache(maxsize=None)
def _build(P, C, NP):
    info = plsc.get_sparse_core_info()
    NC, NS = info.num_cores, info.num_subcores
    NW = NC * NS
    assert P % (NW * NP) == 0 and C % _LANES == 0
    rows_per_worker = P // NW
    nblocks = rows_per_worker // NP
    assert nblocks % 2 == 0 and nblocks >= 4
    nchunks = C // _LANES
    BLK = NP * C

    mesh = plsc.VectorSubcoreMesh(core_axis_name="c", subcore_axis_name="s")

    @functools.partial(
        pl.kernel,
        mesh=mesh,
        compiler_params=pltpu.CompilerParams(needs_layout_passes=False),
        out_type=jax.ShapeDtypeStruct((P * C,), jnp.float32),
        scratch_types=[
            pltpu.VMEM((BLK,), jnp.float32),
            pltpu.VMEM((BLK,), jnp.int32),
            pltpu.VMEM((BLK,), jnp.float32),
            pltpu.VMEM((BLK,), jnp.float32),
            pltpu.VMEM((BLK,), jnp.int32),
            pltpu.VMEM((BLK,), jnp.float32),
            pltpu.SemaphoreType.DMA,
            pltpu.SemaphoreType.DMA,
            pltpu.SemaphoreType.DMA,
            pltpu.SemaphoreType.DMA,
        ],
    )
    def k(img_hbm, perm_hbm, out_hbm,
          img0, perm0, out0, img1, perm1, out1,
          semi0, semi1, semo0, semo1):
        wid = lax.axis_index("s") * NC + lax.axis_index("c")
        base = wid * rows_per_worker * C
        bufs = ((img0, perm0, out0, semi0, semo0),
                (img1, perm1, out1, semi1, semo1))

        def start_in(b, buf):
            img_v, perm_v, _, semi, _ = buf
            e0 = base + b * BLK
            pltpu.async_copy(img_hbm.at[pl.ds(e0, BLK)], img_v, semi)
            pltpu.async_copy(perm_hbm.at[pl.ds(e0, BLK)], perm_v, semi)

        def wait_in(buf):
            img_v, perm_v, _, semi, _ = buf
            pltpu.make_async_copy(img_hbm.at[pl.ds(base, BLK)], img_v, semi).wait()
            pltpu.make_async_copy(perm_hbm.at[pl.ds(base, BLK)], perm_v, semi).wait()

        def start_out(b, buf):
            out_v, semo = buf[2], buf[4]
            e0 = base + b * BLK
            pltpu.async_copy(out_v, out_hbm.at[pl.ds(e0, BLK)], semo)

        def wait_out(buf):
            out_v, semo = buf[2], buf[4]
            pltpu.make_async_copy(out_v, out_hbm.at[pl.ds(base, BLK)], semo).wait()

        def compute(buf):
            img_v, perm_v, out_v = buf[0], buf[1], buf[2]

            def pix(i, c):
                basei = i * C
                off = jnp.full((_LANES,), basei, jnp.int32)
                for j in range(nchunks):
                    sl = pl.ds(basei + j * _LANES, _LANES)
                    out_v[sl] = plsc.load_gather(img_v, [perm_v[sl] + off])
                return c

            lax.fori_loop(0, NP, pix, 0)

        start_in(0, bufs[0])

        @pl.loop(0, nblocks, step=2)
        def outer(b0):
            for r in range(2):
                b = b0 + r
                buf = bufs[r]

                @pl.when(b + 1 < nblocks)
                def _():
                    start_in(b + 1, bufs[1 - r])

                @pl.when(b >= 2)
                def _():
                    wait_out(buf)

                wait_in(buf)
                compute(buf)
                start_out(b, buf)

        wait_out(bufs[0])
        wait_out(bufs[1])

    return k


def kernel(image, perm):
    W, H, C = image.shape
    P = W * H
    img1 = image.reshape(P * C)
    perm1 = perm.reshape(P * C)
    out1 = _build(P, C, 48)(img1, perm1)
    return out1.reshape(W, H, C)
